# baseline (speedup 1.0000x reference)
import functools

import jax
import jax.numpy as jnp
from jax import lax
from jax.experimental import pallas as pl
from jax.experimental.pallas import tpu as pltpu
from jax.experimental.pallas import tpu_sc as plsc

N_NODES = 50000
E = 1600000
E_SF = 400000
D_EDGE = 16

NC = 2
NS = 16
NW = NC * NS

CHUNK = 80
E_PER_W = E // NW
A_CHUNKS = E_PER_W // CHUNK
SF_PER_CORE = E_SF // NC
B_CHUNKS = 156
SF_MAIN = NS * B_CHUNKS * CHUNK

N_PAD = 50048
NODES_PER_TILE = N_PAD // NS


NBUF = 5


def _pipelined_scatter(h_idx1, h_idx2, h_rows, base, nlaps, nbuf, acc1, acc2,
                       idx1b, idx2b, rowsb, semr, sems):
    def reads(lap, b):
        off = base + (lap * nbuf + b) * CHUNK
        pltpu.async_copy(h_idx1.at[pl.ds(off, CHUNK)], idx1b[b], semr[b])
        pltpu.async_copy(h_idx2.at[pl.ds(off, CHUNK)], idx2b[b], semr[b])
        pltpu.async_copy(h_rows.at[pl.ds(off, CHUNK)], rowsb[b], semr[b])

    def wait_reads(lap, b):
        off = base + (lap * nbuf + b) * CHUNK
        pltpu.make_async_copy(h_idx1.at[pl.ds(off, CHUNK)], idx1b[b],
                              semr[b]).wait()
        pltpu.make_async_copy(h_idx2.at[pl.ds(off, CHUNK)], idx2b[b],
                              semr[b]).wait()
        pltpu.make_async_copy(h_rows.at[pl.ds(off, CHUNK)], rowsb[b],
                              semr[b]).wait()

    for b in range(nbuf):
        reads(0, b)

    def lap_body(lap, carry):
        for b in range(nbuf):
            wait_reads(lap, b)
            pltpu.async_copy(rowsb[b], acc1.at[idx1b[b]], sems[b], add=True)
            pltpu.async_copy(rowsb[b], acc2.at[idx2b[b]], sems[b], add=True)
        for b in range(nbuf):
            pltpu.make_async_copy(rowsb[b], acc1.at[idx1b[b]], sems[b]).wait()
            pltpu.make_async_copy(rowsb[b], acc2.at[idx2b[b]], sems[b]).wait()

            @pl.when(lap + 1 < nlaps)
            def _():
                reads(lap + 1, b)

        return carry

    lax.fori_loop(0, nlaps, lap_body, 0)


def _sc_body(fut_hbm, past_hbm, eattr_hbm, early_hbm, later_hbm, sfattr_hbm,
             zeros_hbm, out_hbm, *scratch):
    idx1b = scratch[0:NBUF]
    idx2b = scratch[NBUF:2 * NBUF]
    rowsb = scratch[2 * NBUF:3 * NBUF]
    acc_a, acc_b = scratch[3 * NBUF], scratch[3 * NBUF + 1]
    semr = scratch[3 * NBUF + 2:4 * NBUF + 2]
    sems = scratch[4 * NBUF + 2:5 * NBUF + 2]

    c = lax.axis_index("c")
    s = lax.axis_index("s")
    wid = s * NC + c
    nsl = pl.ds(s * NODES_PER_TILE, NODES_PER_TILE)

    pltpu.sync_copy(zeros_hbm.at[nsl], acc_a.at[nsl])
    pltpu.sync_copy(zeros_hbm.at[nsl], acc_b.at[nsl])
    plsc.subcore_barrier()

    _pipelined_scatter(fut_hbm, past_hbm, eattr_hbm, wid * E_PER_W,
                       A_CHUNKS // NBUF, NBUF, acc_a, acc_b,
                       idx1b, idx2b, rowsb, semr, sems)
    plsc.subcore_barrier()

    pltpu.sync_copy(acc_a.at[nsl], out_hbm.at[0, c, nsl])
    pltpu.sync_copy(acc_b.at[nsl], out_hbm.at[2, c, nsl])
    pltpu.sync_copy(zeros_hbm.at[nsl], acc_a.at[nsl])
    plsc.subcore_barrier()

    base_sf = c * SF_PER_CORE + s * (B_CHUNKS * CHUNK)
    _pipelined_scatter(early_hbm, later_hbm, sfattr_hbm, base_sf,
                       B_CHUNKS // 4, 4, acc_a, acc_a,
                       idx1b, idx2b, rowsb, semr, sems)

    @pl.when(s < (SF_PER_CORE - SF_MAIN) // CHUNK)
    def _():
        off = c * SF_PER_CORE + SF_MAIN + s * CHUNK
        pltpu.sync_copy(early_hbm.at[pl.ds(off, CHUNK)], idx1b[0])
        pltpu.sync_copy(later_hbm.at[pl.ds(off, CHUNK)], idx2b[0])
        pltpu.sync_copy(sfattr_hbm.at[pl.ds(off, CHUNK)], rowsb[0])
        pltpu.sync_copy(rowsb[0], acc_a.at[idx1b[0]], add=True)
        pltpu.sync_copy(rowsb[0], acc_a.at[idx2b[0]], add=True)

    plsc.subcore_barrier()
    pltpu.sync_copy(acc_a.at[nsl], out_hbm.at[1, c, nsl])


@jax.jit
def _sc_aggregate(fut, past, eattr, early, later, sfattr, zeros):
    mesh = plsc.VectorSubcoreMesh(core_axis_name="c", subcore_axis_name="s")
    return pl.kernel(
        _sc_body,
        out_type=jax.ShapeDtypeStruct((3, NC, N_PAD, D_EDGE), jnp.float32),
        mesh=mesh,
        compiler_params=pltpu.CompilerParams(use_tc_tiling_on_sc=False),
        scratch_types=(
            [pltpu.VMEM((CHUNK,), jnp.int32) for _ in range(2 * NBUF)]
            + [pltpu.VMEM((CHUNK, D_EDGE), jnp.float32) for _ in range(NBUF)]
            + [pltpu.VMEM_SHARED((N_PAD, D_EDGE), jnp.float32),
               pltpu.VMEM_SHARED((N_PAD, D_EDGE), jnp.float32)]
            + [pltpu.SemaphoreType.DMA for _ in range(2 * NBUF)]
        ),
    )(fut, past, eattr, early, later, sfattr, zeros)


def _lin_body(x_ref, o_ref):
    x = x_ref[...]
    y = x.T
    z = y.reshape(y.shape[0] // 8, 8, 16)
    o_ref[...] = jnp.concatenate([z[:, j, :] for j in range(8)], axis=1)


def _linearize(xt, blk):
    m = xt.shape[1]
    return pl.pallas_call(
        _lin_body,
        grid=(m // blk,),
        in_specs=[pl.BlockSpec((16, blk), lambda i: (0, i))],
        out_specs=pl.BlockSpec((blk // 8, 128), lambda i: (i, 0)),
        out_shape=jax.ShapeDtypeStruct((m // 8, 128), jnp.float32),
    )(xt)


ROWS_BLK = 2000


def _mlp_body(p_ref, w1_ref, b1_ref, w2_ref, b2_ref, o_ref):
    p = p_ref[...]
    x = jnp.concatenate(
        [p[0, 0] + p[0, 1], p[1, 0] + p[1, 1], p[2, 0] + p[2, 1]], axis=1)
    h = jnp.maximum(
        jnp.dot(x, w1_ref[...], preferred_element_type=jnp.float32)
        + b1_ref[...], 0.0)
    o_ref[...] = (
        jnp.dot(h, w2_ref[...], preferred_element_type=jnp.float32)
        + b2_ref[...])


@jax.jit
def _mlp(partials, W1, b1, W2, b2):
    n_blocks = N_NODES // ROWS_BLK
    d_hid = W1.shape[1]
    d_out = W2.shape[1]
    return pl.pallas_call(
        _mlp_body,
        grid=(n_blocks,),
        in_specs=[
            pl.BlockSpec((3, NC, ROWS_BLK, D_EDGE), lambda i: (0, 0, i, 0)),
            pl.BlockSpec((W1.shape[0], d_hid), lambda i: (0, 0)),
            pl.BlockSpec((1, d_hid), lambda i: (0, 0)),
            pl.BlockSpec((d_hid, d_out), lambda i: (0, 0)),
            pl.BlockSpec((1, d_out), lambda i: (0, 0)),
        ],
        out_specs=pl.BlockSpec((ROWS_BLK, d_out), lambda i: (i, 0)),
        out_shape=jax.ShapeDtypeStruct((N_NODES, d_out), jnp.float32),
    )(partials, W1, b1, W2, b2)


def kernel(edge_index, edge_attr, num_nodes, same_frame_edge_index,
           same_frame_edge_attr, W1, b1, W2, b2):
    del num_nodes
    ei = edge_index.astype(jnp.int32)
    sfi = same_frame_edge_index.astype(jnp.int32)
    zeros = jnp.zeros((N_PAD, D_EDGE), jnp.float32)
    eattr_lin = _linearize(edge_attr.T, 2560).reshape(E, D_EDGE)
    sfattr_lin = _linearize(same_frame_edge_attr.T, 640).reshape(E_SF, D_EDGE)
    partials = _sc_aggregate(ei[1], ei[0], eattr_lin, sfi[0], sfi[1],
                             sfattr_lin, zeros)
    return _mlp(partials, W1, b1.reshape(1, -1), W2, b2.reshape(1, -1))


# speedup vs baseline: 1.5843x; 1.5843x over previous
import functools

import jax
import jax.numpy as jnp
from jax import lax
from jax.experimental import pallas as pl
from jax.experimental.pallas import tpu as pltpu
from jax.experimental.pallas import tpu_sc as plsc

N_NODES = 50000
E = 1600000
E_SF = 400000
D_EDGE = 16

NC = 2
NS = 16
NW = NC * NS

CHUNK = 80
E_PER_W = E // NW
A_CHUNKS = E_PER_W // CHUNK
SF_PER_CORE = E_SF // NC
B_CHUNKS = 156
SF_MAIN = NS * B_CHUNKS * CHUNK

N_PAD = 50048
NODES_PER_TILE = N_PAD // NS


NBUF = 5


def _pipelined_scatter(h_idx1, h_idx2, h_rows, base, nlaps, nbuf, acc1, acc2,
                       idx1b, idx2b, rowsb, semr, sems):
    def reads(lap, b):
        off = base + (lap * nbuf + b) * CHUNK
        pltpu.async_copy(h_idx1.at[pl.ds(off, CHUNK)], idx1b[b], semr[b])
        pltpu.async_copy(h_idx2.at[pl.ds(off, CHUNK)], idx2b[b], semr[b])
        pltpu.async_copy(h_rows.at[pl.ds(off, CHUNK)], rowsb[b], semr[b])

    def wait_reads(lap, b):
        off = base + (lap * nbuf + b) * CHUNK
        pltpu.make_async_copy(h_idx1.at[pl.ds(off, CHUNK)], idx1b[b],
                              semr[b]).wait()
        pltpu.make_async_copy(h_idx2.at[pl.ds(off, CHUNK)], idx2b[b],
                              semr[b]).wait()
        pltpu.make_async_copy(h_rows.at[pl.ds(off, CHUNK)], rowsb[b],
                              semr[b]).wait()

    for b in range(nbuf):
        reads(0, b)

    def lap_body(lap, carry):
        for b in range(nbuf):
            wait_reads(lap, b)
            pltpu.async_copy(rowsb[b], acc1.at[idx1b[b]], sems[b], add=True)
            pltpu.async_copy(rowsb[b], acc2.at[idx2b[b]], sems[b], add=True)
        for b in range(nbuf):
            pltpu.make_async_copy(rowsb[b], acc1.at[idx1b[b]], sems[b]).wait()
            pltpu.make_async_copy(rowsb[b], acc2.at[idx2b[b]], sems[b]).wait()

            @pl.when(lap + 1 < nlaps)
            def _():
                reads(lap + 1, b)

        return carry

    lax.fori_loop(0, nlaps, lap_body, 0)


def _sc_a_body(fut_hbm, past_hbm, eattr_hbm, zeros_hbm, out_hbm, *scratch):
    idx1b = scratch[0:NBUF]
    idx2b = scratch[NBUF:2 * NBUF]
    rowsb = scratch[2 * NBUF:3 * NBUF]
    acc_a, acc_b = scratch[3 * NBUF], scratch[3 * NBUF + 1]
    semr = scratch[3 * NBUF + 2:4 * NBUF + 2]
    sems = scratch[4 * NBUF + 2:5 * NBUF + 2]

    c = lax.axis_index("c")
    s = lax.axis_index("s")
    wid = s * NC + c
    nsl = pl.ds(s * NODES_PER_TILE, NODES_PER_TILE)

    pltpu.sync_copy(zeros_hbm.at[nsl], acc_a.at[nsl])
    pltpu.sync_copy(zeros_hbm.at[nsl], acc_b.at[nsl])
    plsc.subcore_barrier()

    _pipelined_scatter(fut_hbm, past_hbm, eattr_hbm, wid * E_PER_W,
                       A_CHUNKS // NBUF, NBUF, acc_a, acc_b,
                       idx1b, idx2b, rowsb, semr, sems)
    plsc.subcore_barrier()
    pltpu.sync_copy(acc_a.at[nsl], out_hbm.at[0, c, nsl])
    pltpu.sync_copy(acc_b.at[nsl], out_hbm.at[1, c, nsl])


def _sc_b_body(early_hbm, later_hbm, sfattr_hbm, zeros_hbm, out_hbm,
               *scratch):
    idx1b = scratch[0:NBUF]
    idx2b = scratch[NBUF:2 * NBUF]
    rowsb = scratch[2 * NBUF:3 * NBUF]
    acc_a = scratch[3 * NBUF]
    semr = scratch[3 * NBUF + 1:4 * NBUF + 1]
    sems = scratch[4 * NBUF + 1:5 * NBUF + 1]

    c = lax.axis_index("c")
    s = lax.axis_index("s")
    nsl = pl.ds(s * NODES_PER_TILE, NODES_PER_TILE)

    pltpu.sync_copy(zeros_hbm.at[nsl], acc_a.at[nsl])
    plsc.subcore_barrier()

    base_sf = c * SF_PER_CORE + s * (B_CHUNKS * CHUNK)
    _pipelined_scatter(early_hbm, later_hbm, sfattr_hbm, base_sf,
                       B_CHUNKS // 4, 4, acc_a, acc_a,
                       idx1b, idx2b, rowsb, semr, sems)

    @pl.when(s < (SF_PER_CORE - SF_MAIN) // CHUNK)
    def _():
        off = c * SF_PER_CORE + SF_MAIN + s * CHUNK
        pltpu.sync_copy(early_hbm.at[pl.ds(off, CHUNK)], idx1b[0])
        pltpu.sync_copy(later_hbm.at[pl.ds(off, CHUNK)], idx2b[0])
        pltpu.sync_copy(sfattr_hbm.at[pl.ds(off, CHUNK)], rowsb[0])
        pltpu.sync_copy(rowsb[0], acc_a.at[idx1b[0]], add=True)
        pltpu.sync_copy(rowsb[0], acc_a.at[idx2b[0]], add=True)

    plsc.subcore_barrier()
    pltpu.sync_copy(acc_a.at[nsl], out_hbm.at[c, nsl])


def _sc_params():
    return dict(
        mesh=plsc.VectorSubcoreMesh(core_axis_name="c", subcore_axis_name="s"),
        compiler_params=pltpu.CompilerParams(use_tc_tiling_on_sc=False),
    )


def _sc_scratch(n_accs):
    return (
        [pltpu.VMEM((CHUNK,), jnp.int32) for _ in range(2 * NBUF)]
        + [pltpu.VMEM((CHUNK, D_EDGE), jnp.float32) for _ in range(NBUF)]
        + [pltpu.VMEM_SHARED((N_PAD, D_EDGE), jnp.float32)] * n_accs
        + [pltpu.SemaphoreType.DMA for _ in range(2 * NBUF)]
    )


@jax.jit
def _sc_phase_a(fut, past, eattr, zeros):
    return pl.kernel(
        _sc_a_body,
        out_type=jax.ShapeDtypeStruct((2, NC, N_PAD, D_EDGE), jnp.float32),
        scratch_types=_sc_scratch(2),
        **_sc_params(),
    )(fut, past, eattr, zeros)


@jax.jit
def _sc_phase_b(early, later, sfattr, zeros):
    return pl.kernel(
        _sc_b_body,
        out_type=jax.ShapeDtypeStruct((NC, N_PAD, D_EDGE), jnp.float32),
        scratch_types=_sc_scratch(1),
        **_sc_params(),
    )(early, later, sfattr, zeros)


def _lin_body(x_ref, o_ref):
    x = x_ref[...]
    y = x.T
    z = y.reshape(y.shape[0] // 8, 8, 16)
    o_ref[...] = jnp.concatenate([z[:, j, :] for j in range(8)], axis=1)


def _linearize(xt, blk):
    m = xt.shape[1]
    return pl.pallas_call(
        _lin_body,
        grid=(m // blk,),
        in_specs=[pl.BlockSpec((16, blk), lambda i: (0, i))],
        out_specs=pl.BlockSpec((blk // 8, 128), lambda i: (i, 0)),
        out_shape=jax.ShapeDtypeStruct((m // 8, 128), jnp.float32),
    )(xt)


ROWS_BLK = 2000


def _mlp_body(pa_ref, pb_ref, w1_ref, b1_ref, w2_ref, b2_ref, o_ref):
    pa = pa_ref[...]
    pb = pb_ref[...]
    x = jnp.concatenate(
        [pa[0, 0] + pa[0, 1], pb[0] + pb[1], pa[1, 0] + pa[1, 1]], axis=1)
    h = jnp.maximum(
        jnp.dot(x, w1_ref[...], preferred_element_type=jnp.float32)
        + b1_ref[...], 0.0)
    o_ref[...] = (
        jnp.dot(h, w2_ref[...], preferred_element_type=jnp.float32)
        + b2_ref[...])


@jax.jit
def _mlp(pa, pb, W1, b1, W2, b2):
    n_blocks = N_NODES // ROWS_BLK
    d_hid = W1.shape[1]
    d_out = W2.shape[1]
    return pl.pallas_call(
        _mlp_body,
        grid=(n_blocks,),
        in_specs=[
            pl.BlockSpec((2, NC, ROWS_BLK, D_EDGE), lambda i: (0, 0, i, 0)),
            pl.BlockSpec((NC, ROWS_BLK, D_EDGE), lambda i: (0, i, 0)),
            pl.BlockSpec((W1.shape[0], d_hid), lambda i: (0, 0)),
            pl.BlockSpec((1, d_hid), lambda i: (0, 0)),
            pl.BlockSpec((d_hid, d_out), lambda i: (0, 0)),
            pl.BlockSpec((1, d_out), lambda i: (0, 0)),
        ],
        out_specs=pl.BlockSpec((ROWS_BLK, d_out), lambda i: (i, 0)),
        out_shape=jax.ShapeDtypeStruct((N_NODES, d_out), jnp.float32),
    )(pa, pb, W1, b1, W2, b2)


def kernel(edge_index, edge_attr, num_nodes, same_frame_edge_index,
           same_frame_edge_attr, W1, b1, W2, b2):
    del num_nodes
    ei = edge_index.astype(jnp.int32)
    sfi = same_frame_edge_index.astype(jnp.int32)
    zeros = jnp.zeros((N_PAD, D_EDGE), jnp.float32)
    eattr_lin = _linearize(edge_attr.T, 12800).reshape(E, D_EDGE)
    sfattr_lin = _linearize(same_frame_edge_attr.T, 3200).reshape(E_SF,
                                                                  D_EDGE)
    pa = _sc_phase_a(ei[1], ei[0], eattr_lin, zeros)
    pb = _sc_phase_b(sfi[0], sfi[1], sfattr_lin, zeros)
    return _mlp(pa, pb, W1, b1.reshape(1, -1), W2, b2.reshape(1, -1))
